# baseline (speedup 1.0000x reference)
import jax
import jax.numpy as jnp
from jax import lax
from jax.experimental import pallas as pl
from jax.experimental.pallas import tpu as pltpu

T = 8192
D = 2048
E = 16
K = 640
TB = 1024


def _scores_body(h_ref, e_ref, s_ref):
    s_ref[...] = lax.dot_general(e_ref[...], h_ref[...],
                                 (((1,), (1,)), ((), ())),
                                 preferred_element_type=jnp.float32)


def _topk_body(s_ref, ew_ref, ti_ref, cw_ref):
    v = s_ref[...]
    pos = lax.broadcasted_iota(jnp.int32, (E, T), 1)
    idx = pos

    def cond(c):
        return c[0] <= T

    def body(c):
        k, j, v, idx = c
        up_v = pltpu.roll(v, T - j, 1)
        dn_v = pltpu.roll(v, j, 1)
        up_i = pltpu.roll(idx, T - j, 1)
        dn_i = pltpu.roll(idx, j, 1)
        is_lo = (pos & j) == 0
        pv = jnp.where(is_lo, up_v, dn_v)
        pi = jnp.where(is_lo, up_i, dn_i)
        desc = (pos & k) == 0
        better = (v > pv) | ((v == pv) & (idx < pi))
        keep = better == (desc == is_lo)
        v = jnp.where(keep, v, pv)
        idx = jnp.where(keep, idx, pi)
        j2 = j // 2
        k2 = jnp.where(j2 == 0, k * 2, k)
        j2 = jnp.where(j2 == 0, k, j2)
        return (k2, j2, v, idx)

    _, _, v, idx = lax.while_loop(
        cond, body, (jnp.int32(2), jnp.int32(1), v, idx))

    top_v = v[:, :K]
    m = top_v[:, :1]
    ex = jnp.exp(top_v - m)
    denom = jnp.sum(ex, axis=1, keepdims=True)
    ew_ref[...] = ex / denom
    ti_ref[...] = idx[:, :K]

    tau = v[:, K - 1:K]
    tie = idx[:, K - 1:K]
    s = s_ref[...]
    sel = (s > tau) | ((s == tau) & (pos <= tie))
    cw_ref[...] = jnp.where(sel, jnp.exp(s - m) / denom, 0.0)


def _normalize(x, axis=-1, eps=1e-12):
    n = jnp.linalg.norm(x, axis=axis, keepdims=True)
    return x / jnp.maximum(n, eps)


def kernel(hidden_states, expert_embeddings):
    hidden_states = _normalize(hidden_states)
    expert_embeddings = _normalize(expert_embeddings)
    scores = pl.pallas_call(
        _scores_body,
        grid=(T // TB,),
        in_specs=[pl.BlockSpec((TB, D), lambda i: (i, 0)),
                  pl.BlockSpec((E, D), lambda i: (0, 0))],
        out_specs=pl.BlockSpec((E, TB), lambda i: (0, i)),
        out_shape=jax.ShapeDtypeStruct((E, T), jnp.float32),
    )(hidden_states, expert_embeddings)

    ew, ti, cw_t = pl.pallas_call(
        _topk_body,
        out_shape=(jax.ShapeDtypeStruct((E, K), jnp.float32),
                   jax.ShapeDtypeStruct((E, K), jnp.int32),
                   jax.ShapeDtypeStruct((E, T), jnp.float32)),
    )(scores)
    return (ew[..., None], ti, cw_t.T)


# speedup vs baseline: 1.4102x; 1.4102x over previous
import jax
import jax.numpy as jnp
from jax import lax
from jax.experimental import pallas as pl
from jax.experimental.pallas import tpu as pltpu
from jax.experimental.pallas import tpu_sc as plsc

T = 8192
D = 2048
E = 16
K = 640
TB = 1024
NSEL = 1024
INT_MIN_I = -2147483648


def _scores_body(h_ref, e_ref, s_ref):
    s_ref[...] = lax.dot_general(e_ref[...], h_ref[...],
                                 (((1,), (1,)), ((), ())),
                                 preferred_element_type=jnp.float32)


def _sc_select_body(s_hbm, ov_hbm, oi_hbm, row_v, key_v, actA, actB, ov, oi):
    c = lax.axis_index("c")
    s = lax.axis_index("s")
    wid = s * 2 + c
    lane = lax.iota(jnp.int32, 16)
    zeros16 = jnp.zeros((16,), jnp.int32)

    @pl.when(wid < E)
    def _():
        pltpu.sync_copy(s_hbm.at[wid], row_v)

        def kb(i, carry):
            v = row_v[pl.ds(i * 16, 16)]
            b = lax.bitcast_convert_type(v, jnp.int32)
            key_v[pl.ds(i * 16, 16)] = jnp.where(
                b < 0, b ^ jnp.int32(0x7FFFFFFF), b)
            return carry
        lax.fori_loop(0, T // 16, kb, jnp.int32(0))

        def cnt31(i, acc):
            m = key_v[pl.ds(i * 16, 16)]
            return acc + jnp.where(m >= 0, 1, 0).astype(jnp.int32)
        c1 = jnp.sum(lax.fori_loop(0, T // 16, cnt31, zeros16))
        take1 = c1 >= K
        above = jnp.where(take1, jnp.int32(0), c1)
        want1 = jnp.where(take1, jnp.int32(1), jnp.int32(0))
        prefix = jnp.where(take1, jnp.int32(INT_MIN_I), jnp.int32(0))

        def cp31(i, off):
            m = key_v[pl.ds(i * 16, 16)]
            bit = jnp.where(m >= 0, 1, 0).astype(jnp.int32)
            msk = bit == want1
            idxv = i * 16 + lane
            plsc.store_compressed(actA.at[pl.ds(off, 16)], idxv, mask=msk)
            return off + jnp.sum(msk.astype(jnp.int32))
        n = lax.fori_loop(0, T // 16, cp31, jnp.int32(0))

        bufs = [actA, actB]
        for b in range(30, -1, -1):
            src = bufs[(30 - b) % 2]
            dst = bufs[(31 - b) % 2]
            nv = (n + 15) // 16

            def cntb(i, acc, n=n, src=src, b=b):
                valid = (i * 16 + lane) < n
                idxs = src[pl.ds(i * 16, 16)]
                mm = plsc.load_gather(key_v, [jnp.where(valid, idxs, 0)])
                bit = (mm >> b) & 1
                return acc + jnp.where(valid & (bit == 1), 1, 0).astype(jnp.int32)
            c1 = jnp.sum(lax.fori_loop(0, nv, cntb, zeros16))
            take1 = (above + c1) >= K
            want = jnp.where(take1, jnp.int32(1), jnp.int32(0))
            above = jnp.where(take1, above, above + c1)
            prefix = prefix | (want << b)

            def cpb(i, off, n=n, src=src, dst=dst, b=b, want=want):
                valid = (i * 16 + lane) < n
                idxs = src[pl.ds(i * 16, 16)]
                mm = plsc.load_gather(key_v, [jnp.where(valid, idxs, 0)])
                bit = (mm >> b) & 1
                msk = valid & (bit == want)
                plsc.store_compressed(dst.at[pl.ds(off, 16)], idxs, mask=msk)
                return off + jnp.sum(msk.astype(jnp.int32))
            n = lax.fori_loop(0, nv, cpb, jnp.int32(0))

        m_kt = prefix ^ jnp.int32(INT_MIN_I)

        def pad(i, carry):
            ov[pl.ds(i * 16, 16)] = jnp.full((16,), -jnp.inf, jnp.float32)
            oi[pl.ds(i * 16, 16)] = jnp.full((16,), 0x7FFFFFF, jnp.int32)
            return carry
        lax.fori_loop(0, NSEL // 16, pad, jnp.int32(0))

        def fin(i, offs):
            og, ot = offs
            mm = key_v[pl.ds(i * 16, 16)]
            vv = row_v[pl.ds(i * 16, 16)]
            idxv = i * 16 + lane
            gt = mm > m_kt
            plsc.store_compressed(ov.at[pl.ds(og, 16)], vv, mask=gt)
            plsc.store_compressed(oi.at[pl.ds(og, 16)], idxv, mask=gt)
            eq = mm == m_kt
            rank = ot + plsc.cumsum(eq.astype(jnp.int32)) - 1
            tk = eq & (rank < K)
            plsc.store_compressed(ov.at[pl.ds(ot, 16)], vv, mask=tk)
            plsc.store_compressed(oi.at[pl.ds(ot, 16)], idxv, mask=tk)
            return (og + jnp.sum(gt.astype(jnp.int32)),
                    ot + jnp.sum(tk.astype(jnp.int32)))
        lax.fori_loop(0, T // 16, fin, (jnp.int32(0), above))

        pltpu.sync_copy(ov, ov_hbm.at[wid])
        pltpu.sync_copy(oi, oi_hbm.at[wid])


def _final_body(s_ref, v_ref, i_ref, ew_ref, ti_ref, cw_ref):
    v = v_ref[...]
    idx = i_ref[...]
    pos = lax.broadcasted_iota(jnp.int32, (E, NSEL), 1)

    def cond(c):
        return c[0] <= NSEL

    def body(c):
        k, j, v, idx = c
        up_v = pltpu.roll(v, NSEL - j, 1)
        dn_v = pltpu.roll(v, j, 1)
        up_i = pltpu.roll(idx, NSEL - j, 1)
        dn_i = pltpu.roll(idx, j, 1)
        is_lo = (pos & j) == 0
        pv = jnp.where(is_lo, up_v, dn_v)
        pi = jnp.where(is_lo, up_i, dn_i)
        desc = (pos & k) == 0
        better = (v > pv) | ((v == pv) & (idx < pi))
        keep = better == (desc == is_lo)
        v = jnp.where(keep, v, pv)
        idx = jnp.where(keep, idx, pi)
        j2 = j // 2
        k2 = jnp.where(j2 == 0, k * 2, k)
        j2 = jnp.where(j2 == 0, k, j2)
        return (k2, j2, v, idx)

    _, _, v, idx = lax.while_loop(
        cond, body, (jnp.int32(2), jnp.int32(1), v, idx))

    top_v = v[:, :K]
    m = top_v[:, :1]
    ex = jnp.exp(top_v - m)
    denom = jnp.sum(ex, axis=1, keepdims=True)
    ew_ref[...] = ex / denom
    ti_ref[...] = idx[:, :K]

    tau = v[:, K - 1:K]
    tie = idx[:, K - 1:K]
    s = s_ref[...]
    spos = lax.broadcasted_iota(jnp.int32, (E, T), 1)
    sel = (s > tau) | ((s == tau) & (spos <= tie))
    cw_ref[...] = jnp.where(sel, jnp.exp(s - m) / denom, 0.0)


def _normalize(x, axis=-1, eps=1e-12):
    n = jnp.linalg.norm(x, axis=axis, keepdims=True)
    return x / jnp.maximum(n, eps)


def kernel(hidden_states, expert_embeddings):
    hidden_states = _normalize(hidden_states)
    expert_embeddings = _normalize(expert_embeddings)
    scores = pl.pallas_call(
        _scores_body,
        grid=(T // TB,),
        in_specs=[pl.BlockSpec((TB, D), lambda i: (i, 0)),
                  pl.BlockSpec((E, D), lambda i: (0, 0))],
        out_specs=pl.BlockSpec((E, TB), lambda i: (0, i)),
        out_shape=jax.ShapeDtypeStruct((E, T), jnp.float32),
    )(hidden_states, expert_embeddings)

    mesh = plsc.VectorSubcoreMesh(core_axis_name="c", subcore_axis_name="s")
    sel_v, sel_i = pl.kernel(
        _sc_select_body,
        out_type=(jax.ShapeDtypeStruct((E, NSEL), jnp.float32),
                  jax.ShapeDtypeStruct((E, NSEL), jnp.int32)),
        mesh=mesh,
        compiler_params=pltpu.CompilerParams(needs_layout_passes=False),
        scratch_types=[pltpu.VMEM((T,), jnp.float32),
                       pltpu.VMEM((T,), jnp.int32),
                       pltpu.VMEM((T + 16,), jnp.int32),
                       pltpu.VMEM((T + 16,), jnp.int32),
                       pltpu.VMEM((NSEL,), jnp.float32),
                       pltpu.VMEM((NSEL,), jnp.int32)],
    )(scores)

    ew, ti, cw_t = pl.pallas_call(
        _final_body,
        out_shape=(jax.ShapeDtypeStruct((E, K), jnp.float32),
                   jax.ShapeDtypeStruct((E, K), jnp.int32),
                   jax.ShapeDtypeStruct((E, T), jnp.float32)),
    )(scores, sel_v, sel_i)
    return (ew[..., None], ti, cw_t.T)


# speedup vs baseline: 1.4417x; 1.0224x over previous
import jax
import jax.numpy as jnp
from jax import lax
from jax.experimental import pallas as pl
from jax.experimental.pallas import tpu as pltpu
from jax.experimental.pallas import tpu_sc as plsc

T = 8192
D = 2048
E = 16
K = 640
TB = 1024
NSEL = 1024
INT_MIN_I = -2147483648


def _scores_body(h_ref, e_ref, s_ref):
    s_ref[...] = lax.dot_general(e_ref[...], h_ref[...],
                                 (((1,), (1,)), ((), ())),
                                 preferred_element_type=jnp.float32)


def _sc_select_body(s_hbm, ov_hbm, oi_hbm, row_v, key_v, actA, actB, ov, oi):
    c = lax.axis_index("c")
    s = lax.axis_index("s")
    wid = s * 2 + c
    lane = lax.iota(jnp.int32, 16)
    zeros16 = jnp.zeros((16,), jnp.int32)
    U = 4

    @pl.when(wid < E)
    def _():
        pltpu.sync_copy(s_hbm.at[wid], row_v)

        def kb(i, acc):
            for u in range(U):
                o = (i * U + u) * 16
                v = row_v[pl.ds(o, 16)]
                b = lax.bitcast_convert_type(v, jnp.int32)
                m = jnp.where(b < 0, b ^ jnp.int32(0x7FFFFFFF), b)
                key_v[pl.ds(o, 16)] = m
                acc = acc + jnp.where(m >= 0, 1, 0).astype(jnp.int32)
            return acc
        c1 = jnp.sum(lax.fori_loop(0, T // 16 // U, kb, zeros16))

        take1 = c1 >= K
        above = jnp.where(take1, jnp.int32(0), c1)
        want1 = jnp.where(take1, jnp.int32(1), jnp.int32(0))
        prefix = jnp.where(take1, jnp.int32(INT_MIN_I), jnp.int32(0))

        def cp31(i, carry):
            off_v, acc = carry
            for u in range(U):
                o = (i * U + u) * 16
                m = key_v[pl.ds(o, 16)]
                msk = jnp.where(m >= 0, 1, 0).astype(jnp.int32) == want1
                cs = plsc.cumsum(msk.astype(jnp.int32))
                plsc.store_scatter(actA, [off_v + cs - 1], o + lane, mask=msk)
                acc = acc + jnp.where(
                    msk & (((m >> 30) & 1) == 1), 1, 0).astype(jnp.int32)
                off_v = off_v + plsc.all_reduce_population_count(msk)
            return (off_v, acc)
        off_v, acc = lax.fori_loop(0, T // 16 // U, cp31, (zeros16, zeros16))
        n = jnp.max(off_v)
        c1 = jnp.sum(acc)

        bufs = [actA, actB]
        for b in range(30, -1, -1):
            src = bufs[(30 - b) % 2]
            dst = bufs[(31 - b) % 2]

            take1 = (above + c1) >= K
            want = jnp.where(take1, jnp.int32(1), jnp.int32(0))
            above = jnp.where(take1, above, above + c1)
            prefix = prefix | (want << b)
            nv = (n + 15) // 16

            def cpb(i, carry, n=n, src=src, dst=dst, b=b, want=want):
                off_v, acc = carry
                valid = (i * 16 + lane) < n
                idxs = src[pl.ds(i * 16, 16)]
                mm = plsc.load_gather(key_v, [jnp.where(valid, idxs, 0)])
                bit = (mm >> b) & 1
                msk = valid & (bit == want)
                cs = plsc.cumsum(msk.astype(jnp.int32))
                plsc.store_scatter(dst, [off_v + cs - 1], idxs, mask=msk)
                nxtb = (mm >> (b - 1)) & 1 if b > 0 else bit
                acc = acc + jnp.where(msk & (nxtb == 1), 1, 0).astype(jnp.int32)
                return (off_v + plsc.all_reduce_population_count(msk), acc)
            off_v, acc = lax.fori_loop(0, nv, cpb, (zeros16, zeros16))
            n = jnp.max(off_v)
            c1 = jnp.sum(acc)

        m_kt = prefix ^ jnp.int32(INT_MIN_I)

        def pad(i, carry):
            for u in range(U):
                o = (i * U + u) * 16
                ov[pl.ds(o, 16)] = jnp.full((16,), -jnp.inf, jnp.float32)
                oi[pl.ds(o, 16)] = jnp.full((16,), 0x7FFFFFF, jnp.int32)
            return carry
        lax.fori_loop(0, NSEL // 16 // U, pad, jnp.int32(0))

        def fin(i, carry):
            og_v, ot_v = carry
            for u in range(U):
                o = (i * U + u) * 16
                mm = key_v[pl.ds(o, 16)]
                vv = row_v[pl.ds(o, 16)]
                idxv = o + lane
                gt = mm > m_kt
                dg = og_v + plsc.cumsum(gt.astype(jnp.int32)) - 1
                plsc.store_scatter(ov, [dg], vv, mask=gt)
                plsc.store_scatter(oi, [dg], idxv, mask=gt)
                og_v = og_v + plsc.all_reduce_population_count(gt)
                eq = mm == m_kt
                de = ot_v + plsc.cumsum(eq.astype(jnp.int32)) - 1
                tk = eq & (de < K)
                plsc.store_scatter(ov, [de], vv, mask=tk)
                plsc.store_scatter(oi, [de], idxv, mask=tk)
                ot_v = ot_v + plsc.all_reduce_population_count(eq)
            return (og_v, ot_v)
        lax.fori_loop(0, T // 16 // U, fin, (zeros16, zeros16 + above))

        pltpu.sync_copy(ov, ov_hbm.at[wid])
        pltpu.sync_copy(oi, oi_hbm.at[wid])


def _final_body(s_ref, v_ref, i_ref, ew_ref, ti_ref, cw_ref):
    v = v_ref[...]
    idx = i_ref[...]
    pos = lax.broadcasted_iota(jnp.int32, (E, NSEL), 1)

    k = 2
    while k <= NSEL:
        j = k // 2
        while j >= 1:
            up_v = jnp.roll(v, -j, 1)
            dn_v = jnp.roll(v, j, 1)
            up_i = jnp.roll(idx, -j, 1)
            dn_i = jnp.roll(idx, j, 1)
            is_lo = (pos & j) == 0
            pv = jnp.where(is_lo, up_v, dn_v)
            pi = jnp.where(is_lo, up_i, dn_i)
            desc = (pos & k) == 0
            better = (v > pv) | ((v == pv) & (idx < pi))
            keep = better == (desc == is_lo)
            v = jnp.where(keep, v, pv)
            idx = jnp.where(keep, idx, pi)
            j //= 2
        k *= 2

    top_v = v[:, :K]
    m = top_v[:, :1]
    ex = jnp.exp(top_v - m)
    denom = jnp.sum(ex, axis=1, keepdims=True)
    ew_ref[...] = ex / denom
    ti_ref[...] = idx[:, :K]

    tau = v[:, K - 1:K]
    tie = idx[:, K - 1:K]
    s = s_ref[...]
    spos = lax.broadcasted_iota(jnp.int32, (E, T), 1)
    sel = (s > tau) | ((s == tau) & (spos <= tie))
    cw_ref[...] = jnp.where(sel, jnp.exp(s - m) / denom, 0.0).T


def _normalize(x, axis=-1, eps=1e-12):
    n = jnp.linalg.norm(x, axis=axis, keepdims=True)
    return x / jnp.maximum(n, eps)


def kernel(hidden_states, expert_embeddings):
    hidden_states = _normalize(hidden_states)
    expert_embeddings = _normalize(expert_embeddings)
    scores = pl.pallas_call(
        _scores_body,
        grid=(T // TB,),
        in_specs=[pl.BlockSpec((TB, D), lambda i: (i, 0)),
                  pl.BlockSpec((E, D), lambda i: (0, 0))],
        out_specs=pl.BlockSpec((E, TB), lambda i: (0, i)),
        out_shape=jax.ShapeDtypeStruct((E, T), jnp.float32),
    )(hidden_states, expert_embeddings)

    mesh = plsc.VectorSubcoreMesh(core_axis_name="c", subcore_axis_name="s")
    sel_v, sel_i = pl.kernel(
        _sc_select_body,
        out_type=(jax.ShapeDtypeStruct((E, NSEL), jnp.float32),
                  jax.ShapeDtypeStruct((E, NSEL), jnp.int32)),
        mesh=mesh,
        compiler_params=pltpu.CompilerParams(needs_layout_passes=False),
        scratch_types=[pltpu.VMEM((T,), jnp.float32),
                       pltpu.VMEM((T,), jnp.int32),
                       pltpu.VMEM((T + 16,), jnp.int32),
                       pltpu.VMEM((T + 16,), jnp.int32),
                       pltpu.VMEM((NSEL,), jnp.float32),
                       pltpu.VMEM((NSEL,), jnp.int32)],
    )(scores)

    ew, ti, cw_t = pl.pallas_call(
        _final_body,
        out_shape=(jax.ShapeDtypeStruct((E, K), jnp.float32),
                   jax.ShapeDtypeStruct((E, K), jnp.int32),
                   jax.ShapeDtypeStruct((T, E), jnp.float32)),
    )(scores, sel_v, sel_i)
    return (ew[..., None], ti, cw_t)
